# baseline (speedup 1.0000x reference)
import jax
import jax.numpy as jnp
import numpy as np
from jax import lax
from jax.experimental import pallas as pl
from jax.experimental.pallas import tpu as pltpu

B = 16384
C = 1000
EPS = 1e-8
CB = 200
NB = C // CB
SUB = 8
SIGN = np.int32(-2147483648)


def _w(m, base):
    df = m + lax.convert_element_type(base, jnp.float32)
    na = lax.bitcast_convert_type(
        lax.bitcast_convert_type(df, jnp.int32) | SIGN, jnp.float32
    )
    return jnp.exp2(na)


NH = 2
BH = B // NH


def _body(y_ref, xt_ref, o_ref, acc_ref, m_ref):
    i = pl.program_id(0)
    j = pl.program_id(1)

    @pl.when(j == 0)
    def _():
        iota = lax.broadcasted_iota(jnp.int32, (SUB, BH), 0)
        m_ref[...] = (iota - y_ref[...]).astype(jnp.float32)
        acc_ref[...] = jnp.zeros_like(acc_ref)

    m = m_ref[...]
    acc_ref[...] += sum(
        _w(m, j * CB + k * SUB) * xt_ref[pl.ds(k * SUB, SUB), :]
        for k in range(CB // SUB)
    )

    @pl.when(j == NB - 1)
    def _():
        s = jnp.sum(acc_ref[...], axis=0, keepdims=True)
        part = jnp.sum(-jnp.log(s + EPS)) * (1.0 / B)

        @pl.when(i == 0)
        def _():
            o_ref[0, 0] = 0.0

        o_ref[0, 0] += part


_call = pl.pallas_call(
    _body,
    grid=(NH, NB),
    in_specs=[
        pl.BlockSpec((1, BH), lambda i, j: (0, i)),
        pl.BlockSpec((CB, BH), lambda i, j: (j, i)),
    ],
    out_specs=pl.BlockSpec(
        (1, 1), lambda i, j: (0, 0), memory_space=pltpu.SMEM
    ),
    out_shape=jax.ShapeDtypeStruct((1, 1), jnp.float32),
    scratch_shapes=[
        pltpu.VMEM((SUB, BH), jnp.float32),
        pltpu.VMEM((SUB, BH), jnp.float32),
    ],
)


def kernel(x, y):
    y2 = y.astype(jnp.int32).reshape(1, B)
    return _call(y2, x.T)[0, 0]
